# baseline (speedup 1.0000x reference)
import functools

import jax
import jax.numpy as jnp
from jax import lax
from jax.experimental import pallas as pl
from jax.experimental.pallas import tpu as pltpu
from jax.experimental.pallas import tpu_sc as plsc

_CHANNELS = 64
_LANES = 16


def _gather_row(idx, table):
    mesh = plsc.VectorSubcoreMesh(core_axis_name="c", subcore_axis_name="s")

    @functools.partial(
        pl.kernel,
        mesh=mesh,
        out_type=jax.ShapeDtypeStruct((1, _CHANNELS), jnp.float32),
        scratch_types=[
            pltpu.VMEM((_LANES,), jnp.int32),
            pltpu.VMEM((1, _CHANNELS), jnp.float32),
        ],
    )
    def body(idx_hbm, table_hbm, out_hbm, idx_v, row_v):
        wid = lax.axis_index("s") * 2 + lax.axis_index("c")

        @pl.when(wid == 0)
        def _():
            pltpu.sync_copy(idx_hbm, idx_v)
            t = idx_v[...][0]
            pltpu.sync_copy(table_hbm.at[pl.ds(t, 1)], row_v)
            pltpu.sync_copy(row_v, out_hbm)

    return body(idx, table)


def kernel(t, pos_enc_weight):
    idx = jnp.full((_LANES,), t, dtype=jnp.int32)
    return _gather_row(idx, pos_enc_weight)


# speedup vs baseline: 1.0552x; 1.0552x over previous
import functools

import jax
import jax.numpy as jnp
from jax import lax
from jax.experimental import pallas as pl
from jax.experimental.pallas import tpu as pltpu
from jax.experimental.pallas import tpu_sc as plsc

_CHANNELS = 64


def _gather_row(idx, table):
    mesh = plsc.ScalarSubcoreMesh(axis_name="c", num_cores=1)

    @functools.partial(
        pl.kernel,
        mesh=mesh,
        out_type=jax.ShapeDtypeStruct((1, _CHANNELS), jnp.float32),
        scratch_types=[
            pltpu.SMEM((1,), jnp.int32),
        ],
    )
    def body(idx_hbm, table_hbm, out_hbm, idx_s):
        pltpu.sync_copy(idx_hbm, idx_s)
        t = idx_s[0]
        pltpu.sync_copy(table_hbm.at[pl.ds(t, 1)], out_hbm)

    return body(idx, table)


def kernel(t, pos_enc_weight):
    idx = jnp.asarray(t, dtype=jnp.int32).reshape(1)
    return _gather_row(idx, pos_enc_weight)


# speedup vs baseline: 1.4565x; 1.3803x over previous
import jax
import jax.numpy as jnp
from jax.experimental import pallas as pl
from jax.experimental.pallas import tpu as pltpu

_CHANNELS = 64
_SUB = 8


def _body(idx_ref, table_ref, out_ref):
    r = idx_ref[0] % _SUB
    rows = table_ref[:, :]
    mask = jax.lax.broadcasted_iota(jnp.int32, (_SUB, _CHANNELS), 0) == r
    out_ref[...] = jnp.sum(jnp.where(mask, rows, 0.0), axis=0, keepdims=True)


def kernel(t, pos_enc_weight):
    idx = jnp.asarray(t, dtype=jnp.int32).reshape(1)
    grid_spec = pltpu.PrefetchScalarGridSpec(
        num_scalar_prefetch=1,
        grid=(1,),
        in_specs=[
            pl.BlockSpec((_SUB, _CHANNELS), lambda i, idx_ref: (idx_ref[0] // _SUB, 0))
        ],
        out_specs=pl.BlockSpec((1, _CHANNELS), lambda i, idx_ref: (0, 0)),
    )
    return pl.pallas_call(
        _body,
        grid_spec=grid_spec,
        out_shape=jax.ShapeDtypeStruct((1, _CHANNELS), jnp.float32),
    )(idx, pos_enc_weight)


# speedup vs baseline: 1.4569x; 1.0002x over previous
import jax
import jax.numpy as jnp
from jax.experimental import pallas as pl
from jax.experimental.pallas import tpu as pltpu

_CHANNELS = 64


def _body(idx_ref, table_ref, out_ref, sem):
    t = idx_ref[0]
    copy = pltpu.make_async_copy(table_ref.at[pl.ds(t, 1)], out_ref, sem)
    copy.start()
    copy.wait()


def kernel(t, pos_enc_weight):
    idx = jnp.asarray(t, dtype=jnp.int32).reshape(1)
    return pl.pallas_call(
        _body,
        in_specs=[
            pl.BlockSpec(memory_space=pltpu.SMEM),
            pl.BlockSpec(memory_space=pl.ANY),
        ],
        out_specs=pl.BlockSpec(memory_space=pl.ANY),
        out_shape=jax.ShapeDtypeStruct((1, _CHANNELS), jnp.float32),
        scratch_shapes=[pltpu.SemaphoreType.DMA],
    )(idx, pos_enc_weight)


# speedup vs baseline: 25.0661x; 17.2057x over previous
import jax
import jax.numpy as jnp
from jax.experimental import pallas as pl
from jax.experimental.pallas import tpu as pltpu

_CHANNELS = 64
_LANES = 128


def _body(idx_ref, tablet_ref, out_ref):
    r = idx_ref[0] % _LANES
    onehot = (
        jax.lax.broadcasted_iota(jnp.int32, (1, _LANES), 1) == r
    ).astype(jnp.float32)
    out_ref[...] = jax.lax.dot_general(
        onehot,
        tablet_ref[...],
        (((1,), (1,)), ((), ())),
        preferred_element_type=jnp.float32,
    )


def kernel(t, pos_enc_weight):
    idx = jnp.asarray(t, dtype=jnp.int32).reshape(1)
    tablet = pos_enc_weight.T
    grid_spec = pltpu.PrefetchScalarGridSpec(
        num_scalar_prefetch=1,
        grid=(1,),
        in_specs=[
            pl.BlockSpec(
                (_CHANNELS, _LANES), lambda i, idx_ref: (0, idx_ref[0] // _LANES)
            )
        ],
        out_specs=pl.BlockSpec((1, _CHANNELS), lambda i, idx_ref: (0, 0)),
    )
    return pl.pallas_call(
        _body,
        grid_spec=grid_spec,
        out_shape=jax.ShapeDtypeStruct((1, _CHANNELS), jnp.float32),
    )(idx, tablet)


# speedup vs baseline: 25.0769x; 1.0004x over previous
import jax
import jax.numpy as jnp
from jax.experimental import pallas as pl
from jax.experimental.pallas import tpu as pltpu

_CHANNELS = 64
_LANES = 128


def _body(idx_ref, tablet_ref, out_ref, win_v, sem):
    t = idx_ref[0]
    base = (t // _LANES) * _LANES
    copy = pltpu.make_async_copy(
        tablet_ref.at[:, pl.ds(base, _LANES)], win_v, sem
    )
    copy.start()
    r = t % _LANES
    onehot = (
        jax.lax.broadcasted_iota(jnp.int32, (1, _LANES), 1) == r
    ).astype(jnp.float32)
    copy.wait()
    out_ref[...] = jax.lax.dot_general(
        onehot,
        win_v[...],
        (((1,), (1,)), ((), ())),
        preferred_element_type=jnp.float32,
    )


def kernel(t, pos_enc_weight):
    idx = jnp.asarray(t, dtype=jnp.int32).reshape(1)
    tablet = pos_enc_weight.T
    return pl.pallas_call(
        _body,
        in_specs=[
            pl.BlockSpec(memory_space=pltpu.SMEM),
            pl.BlockSpec(memory_space=pl.ANY),
        ],
        out_specs=pl.BlockSpec(memory_space=pltpu.VMEM),
        out_shape=jax.ShapeDtypeStruct((1, _CHANNELS), jnp.float32),
        scratch_shapes=[
            pltpu.VMEM((_CHANNELS, _LANES), jnp.float32),
            pltpu.SemaphoreType.DMA,
        ],
    )(idx, tablet)
